# Initial kernel scaffold; baseline (speedup 1.0000x reference)
import jax
import jax.numpy as jnp
from jax.experimental import pallas as pl


def kernel(x, importance, W, b, neuron_emb):
    raise NotImplementedError("write your pallas kernel here")


# speedup vs baseline: 2.4014x; 2.4014x over previous
import jax
import jax.numpy as jnp
from jax.experimental import pallas as pl
from jax.experimental.pallas import tpu as pltpu

_B, _S, _D_MODEL, _D_SPACE = 4, 2048, 1024, 64
_N_FQK, _N_FV, _N_REL, _N_VAL = 512, 256, 128, 64
_N_USED = _N_FQK + _N_FV + _N_REL + _N_VAL
_SEGS = ((0, _N_FQK), (_N_FQK, _N_FV), (768, _N_REL), (896, _N_VAL))
_TK_QK, _TK_V, _TK_REL, _TK_VAL = 64, 32, 16, 3
_S_TILE = 512
_NS = _S // _S_TILE


def _weights_kernel(x_ref, imp_ref, w_ref, b_ref, emb_ref, out_ref, embt_ref):
    bi = pl.program_id(0)
    si = pl.program_id(1)

    @pl.when(jnp.logical_and(bi == 0, si == 0))
    def _():
        e = emb_ref[: _N_USED, :]
        nrm = jnp.sqrt(jnp.sum(e * e, axis=1, keepdims=True))
        embt_ref[...] = (e / (nrm + 1e-12)).T

    @pl.when(jnp.logical_and(bi == 0, si == 0))
    def _():
        out_ref[...] = jnp.zeros_like(out_ref)

    bf = jnp.bfloat16
    x_t = x_ref[0]
    h = (
        jnp.dot(
            x_t.astype(bf),
            w_ref[...].astype(bf),
            preferred_element_type=jnp.float32,
        )
        + b_ref[...]
    )
    logits = jnp.dot(
        h.astype(bf),
        embt_ref[...].astype(bf),
        preferred_element_type=jnp.float32,
    )
    imp_row = imp_ref[0].astype(bf)
    row1 = (
        jax.lax.broadcasted_iota(jnp.int32, (_B, 1), 0) == bi
    ).astype(jnp.float32)
    for o, g in _SEGS:
        l = logits[:, o : o + g]
        m = jnp.max(l, axis=1, keepdims=True)
        e = jnp.exp(l - m)
        s = jnp.sum(e, axis=1, keepdims=True)
        p = e / s
        contrib = jnp.dot(
            imp_row, p.astype(bf), preferred_element_type=jnp.float32
        )
        out_ref[:, o : o + g] += row1 * contrib


def _topk_mask(w, k):
    g = w.shape[1]
    a = w[:, :, None]
    c = w[:, None, :]
    im = jax.lax.broadcasted_iota(jnp.int32, (_B, g, g), 1)
    inn = jax.lax.broadcasted_iota(jnp.int32, (_B, g, g), 2)
    gt = (a > c).astype(jnp.float32)
    tie = jnp.logical_and(a == c, im < inn).astype(jnp.float32)
    rank = jnp.sum(gt + tie, axis=1)
    return (rank < k).astype(jnp.float32)


def _sorted_idx(mask, k):
    g = mask.shape[1]
    im = jax.lax.broadcasted_iota(jnp.int32, (_B, g, g), 1)
    inn = jax.lax.broadcasted_iota(jnp.int32, (_B, g, g), 2)
    tri = (im < inn).astype(jnp.float32)
    asc = jnp.sum(mask[:, :, None] * tri, axis=1)
    j = jax.lax.broadcasted_iota(jnp.int32, (_B, g, k), 2).astype(jnp.float32)
    ni = jax.lax.broadcasted_iota(jnp.int32, (_B, g, k), 1).astype(jnp.float32)
    eq = (asc[:, :, None] == j).astype(jnp.float32) * mask[:, :, None]
    return jnp.sum(eq * ni, axis=1).astype(jnp.int32)


def _renorm(w, mask):
    sw = w * mask
    return sw / (jnp.sum(sw, axis=1, keepdims=True) + 1e-9)


def _route_kernel(w_ref, iqk_ref, iv_ref, rw_ref, vw_ref):
    wall = w_ref[...]
    wqk = wall[:, 0:_N_FQK]
    iqk_ref[...] = _sorted_idx(_topk_mask(wqk, _TK_QK), _TK_QK)
    wv = wall[:, _N_FQK : _N_FQK + _N_FV]
    iv_ref[...] = _sorted_idx(_topk_mask(wv, _TK_V), _TK_V)
    wr = wall[:, 768:896]
    rw_ref[...] = _renorm(wr, _topk_mask(wr, _TK_REL))
    wvl = wall[:, 896:960]
    vw_ref[...] = _renorm(wvl, _topk_mask(wvl, _TK_VAL))


def kernel(x, importance, W, b, neuron_emb):
    imp_row = importance[:, None, :]
    b2 = b.reshape(1, _D_SPACE)

    weights = pl.pallas_call(
        _weights_kernel,
        grid=(_B, _NS),
        in_specs=[
            pl.BlockSpec((1, _S_TILE, _D_MODEL), lambda bi, si: (bi, si, 0)),
            pl.BlockSpec((1, 1, _S_TILE), lambda bi, si: (bi, 0, si)),
            pl.BlockSpec((_D_MODEL, _D_SPACE), lambda bi, si: (0, 0)),
            pl.BlockSpec((1, _D_SPACE), lambda bi, si: (0, 0)),
            pl.BlockSpec(neuron_emb.shape, lambda bi, si: (0, 0)),
        ],
        out_specs=pl.BlockSpec((_B, _N_USED), lambda bi, si: (0, 0)),
        out_shape=jax.ShapeDtypeStruct((_B, _N_USED), jnp.float32),
        scratch_shapes=[pltpu.VMEM((_D_SPACE, _N_USED), jnp.float32)],
    )(x, imp_row, W, b2, neuron_emb)

    iqk, iv, rw, vw = pl.pallas_call(
        _route_kernel,
        out_shape=(
            jax.ShapeDtypeStruct((_B, _TK_QK), jnp.int32),
            jax.ShapeDtypeStruct((_B, _TK_V), jnp.int32),
            jax.ShapeDtypeStruct((_B, _N_REL), jnp.float32),
            jax.ShapeDtypeStruct((_B, _N_VAL), jnp.float32),
        ),
    )(weights)
    return (iqk, iv, rw, rw, vw)
